# Initial kernel scaffold; baseline (speedup 1.0000x reference)
import jax
import jax.numpy as jnp
from jax.experimental import pallas as pl


def kernel(output, target):
    raise NotImplementedError("write your pallas kernel here")


# speedup vs baseline: 12.4345x; 12.4345x over previous
import jax
import jax.numpy as jnp
from jax import lax
from jax.experimental import pallas as pl
from jax.experimental.pallas import tpu as pltpu

_WIN = 4
_LANE = 128


def _loss_kernel(nt, nc, sub, kk, x_ref, t_ref, out_ref, mo_ref, mt_ref,
                 acc_ref):
    step = pl.program_id(0)
    t = step % nt
    ph = t % _WIN
    x = x_ref[0]
    tg = t_ref[0]

    @pl.when(step == 0)
    def _():
        acc_ref[0] = 0.0
        acc_ref[1] = 0.0

    @pl.when(ph == 0)
    def _():
        mo_ref[...] = x
        mt_ref[...] = tg

    @pl.when(ph != 0)
    def _():
        mo_ref[...] = jnp.maximum(mo_ref[...], x)
        mt_ref[...] = jnp.maximum(mt_ref[...], tg)

    @pl.when(ph == _WIN - 1)
    def _():
        d = mo_ref[...] - mt_ref[...]
        acc_ref[0] = acc_ref[0] + jnp.sum(d * d)

    bits = lax.bitcast_convert_type(x, jnp.int32)
    key = jnp.where(bits < 0, bits ^ jnp.int32(0x7FFFFFFF), bits)
    key16 = lax.shift_right_arithmetic(key, 16)

    contrib = jnp.float32(0.0)
    for c in range(nc):
        k16 = key16[c]

        def body(i, lohi):
            lo, hi = lohi
            mid = lax.shift_right_arithmetic(lo + hi, 1)
            cnt = jnp.sum((k16 > mid).astype(jnp.int32))
            pred = cnt < kk
            return jnp.where(pred, lo, mid), jnp.where(pred, mid, hi)

        lo, hi = lax.fori_loop(
            0, 16, body, (jnp.int32(-32769), jnp.int32(32767)))
        d = x[c] - tg[c]
        d2 = d * d
        mhi = k16 > hi
        mband = k16 == hi
        s_hi = jnp.sum(jnp.where(mhi, d2, 0.0))
        s_band = jnp.sum(jnp.where(mband, d2, 0.0))
        g = jnp.sum(mhi.astype(jnp.float32))
        e = jnp.sum(mband.astype(jnp.float32))
        contrib = contrib + s_hi + (jnp.float32(kk) - g) / e * s_band
    acc_ref[1] = acc_ref[1] + contrib

    @pl.when(step == pl.num_programs(0) - 1)
    def _():
        nb = pl.num_programs(0) // nt
        npix = nc * sub * _LANE
        tnorm = jnp.float32(nb * npix * (nt // _WIN))
        snorm = jnp.float32(nb * nt * nc * kk)
        out_ref[0, 0] = acc_ref[0] / tnorm + acc_ref[1] / snorm


def kernel(output, target):
    B, T, C, H, W = output.shape
    hw = H * W
    sub = hw // _LANE
    kk = hw // 10
    xr = output.reshape(B * T, C, sub, _LANE)
    tr = target.reshape(B * T, C, sub, _LANE)
    import functools
    body = functools.partial(_loss_kernel, T, C, sub, kk)
    out = pl.pallas_call(
        body,
        grid=(B * T,),
        in_specs=[pl.BlockSpec((1, C, sub, _LANE), lambda r: (r, 0, 0, 0)),
                  pl.BlockSpec((1, C, sub, _LANE), lambda r: (r, 0, 0, 0))],
        out_specs=pl.BlockSpec(memory_space=pltpu.SMEM),
        out_shape=jax.ShapeDtypeStruct((1, 1), jnp.float32),
        scratch_shapes=[pltpu.VMEM((C, sub, _LANE), jnp.float32),
                        pltpu.VMEM((C, sub, _LANE), jnp.float32),
                        pltpu.SMEM((2,), jnp.float32)],
    )(xr, tr)
    return out[0, 0]


# speedup vs baseline: 31.9224x; 2.5673x over previous
import functools

import jax
import jax.numpy as jnp
from jax import lax
from jax.experimental import pallas as pl
from jax.experimental.pallas import tpu as pltpu

_WIN = 4
_LANE = 128


def _loss_kernel(nw, nc, sub, kk, x_ref, t_ref, out_ref, acc_ref):
    step = pl.program_id(0)
    x = x_ref[0]
    tg = t_ref[0]

    @pl.when(step == 0)
    def _():
        acc_ref[0] = 0.0
        acc_ref[1] = 0.0

    mo = jnp.maximum(jnp.maximum(x[0], x[1]), jnp.maximum(x[2], x[3]))
    mt = jnp.maximum(jnp.maximum(tg[0], tg[1]), jnp.maximum(tg[2], tg[3]))
    dt = mo - mt
    acc_ref[0] = acc_ref[0] + jnp.sum(dt * dt)

    bits = lax.bitcast_convert_type(x, jnp.int32)
    key = jnp.where(bits < 0, bits ^ jnp.int32(0x7FFFFFFF), bits)
    key16 = lax.shift_right_arithmetic(key, 16)

    def body(i, lohi):
        lo, hi = lohi
        mid = lax.shift_right_arithmetic(lo + hi, 1)
        cnt = jnp.sum((key16 > mid).astype(jnp.int32), axis=(2, 3),
                      keepdims=True)
        pred = cnt < kk
        return jnp.where(pred, lo, mid), jnp.where(pred, mid, hi)

    lo0 = jnp.full((_WIN, nc, 1, 1), -32769, jnp.int32)
    hi0 = jnp.full((_WIN, nc, 1, 1), 32767, jnp.int32)
    _, hi = lax.fori_loop(0, 16, body, (lo0, hi0))

    d = x - tg
    d2 = d * d
    mhi = key16 > hi
    mband = key16 == hi
    s_hi = jnp.sum(jnp.where(mhi, d2, 0.0))
    s_band = jnp.sum(jnp.where(mband, d2, 0.0), axis=(2, 3), keepdims=True)
    g = jnp.sum(mhi.astype(jnp.float32), axis=(2, 3), keepdims=True)
    e = jnp.sum(mband.astype(jnp.float32), axis=(2, 3), keepdims=True)
    w = (jnp.float32(kk) - g) / e
    acc_ref[1] = acc_ref[1] + s_hi + jnp.sum(w * s_band)

    @pl.when(step == pl.num_programs(0) - 1)
    def _():
        npix = nc * sub * _LANE
        tnorm = jnp.float32(pl.num_programs(0) * npix)
        snorm = jnp.float32(pl.num_programs(0) * _WIN * nc * kk)
        out_ref[0, 0] = acc_ref[0] / tnorm + acc_ref[1] / snorm


def kernel(output, target):
    B, T, C, H, W = output.shape
    hw = H * W
    sub = hw // _LANE
    kk = hw // 10
    nw = T // _WIN
    xr = output.reshape(B * nw, _WIN, C, sub, _LANE)
    tr = target.reshape(B * nw, _WIN, C, sub, _LANE)
    body = functools.partial(_loss_kernel, nw, C, sub, kk)
    spec = pl.BlockSpec((1, _WIN, C, sub, _LANE),
                        lambda r: (r, 0, 0, 0, 0))
    out = pl.pallas_call(
        body,
        grid=(B * nw,),
        in_specs=[spec, spec],
        out_specs=pl.BlockSpec(memory_space=pltpu.SMEM),
        out_shape=jax.ShapeDtypeStruct((1, 1), jnp.float32),
        scratch_shapes=[pltpu.SMEM((2,), jnp.float32)],
    )(xr, tr)
    return out[0, 0]
